# baseline (device time: 47057 ns/iter reference)
import jax
import jax.numpy as jnp
from jax import lax
from jax.experimental import pallas as pl
from jax.experimental.pallas import tpu as pltpu

V_PER_SHARD = 8192


def kernel(ids, E):
    t = ids.shape[0]
    d = E.shape[1]

    def body(ids_ref, E_ref, out_ref, tok_ref, gsem, send_sem, recv_sem):
        x = lax.axis_index("x")
        y = lax.axis_index("y")
        z = lax.axis_index("z")
        partner = x * 8 + (1 - y) * 4 + z
        base = y * V_PER_SHARD

        barrier = pltpu.get_barrier_semaphore()
        pl.semaphore_signal(
            barrier, inc=1, device_id=partner,
            device_id_type=pl.DeviceIdType.LOGICAL,
        )
        pl.semaphore_wait(barrier, 1)

        def scan(tk, nm):
            rid = ids_ref[tk] - base
            owned = rid.astype(jnp.uint32) < V_PER_SHARD

            @pl.when(owned)
            def _():
                tok_ref[nm] = tk
                pltpu.make_async_copy(
                    E_ref.at[pl.ds(rid, 1)], out_ref.at[pl.ds(tk, 1)], gsem
                ).start()

            return nm + owned.astype(jnp.int32)

        nm = lax.fori_loop(0, t, scan, 0, unroll=8)
        no = t - nm

        src0 = E_ref.at[pl.ds(0, 1)]
        dst0 = out_ref.at[pl.ds(0, 1)]

        def send(j, carry):
            pltpu.make_async_copy(src0, dst0, gsem).wait()
            tk = tok_ref[j]
            row = out_ref.at[pl.ds(tk, 1)]
            pltpu.make_async_remote_copy(
                src_ref=row, dst_ref=row,
                send_sem=send_sem, recv_sem=recv_sem,
                device_id=partner, device_id_type=pl.DeviceIdType.LOGICAL,
            ).start()
            return carry

        lax.fori_loop(0, nm, send, 0)

        def waits_send(j, carry):
            pltpu.make_async_remote_copy(
                src_ref=src0, dst_ref=dst0,
                send_sem=send_sem, recv_sem=recv_sem,
                device_id=partner, device_id_type=pl.DeviceIdType.LOGICAL,
            ).wait_send()
            return carry

        def waits_recv(j, carry):
            pltpu.make_async_remote_copy(
                src_ref=src0, dst_ref=dst0,
                send_sem=send_sem, recv_sem=recv_sem,
                device_id=partner, device_id_type=pl.DeviceIdType.LOGICAL,
            ).wait_recv()
            return carry

        lax.fori_loop(0, nm, waits_send, 0)
        lax.fori_loop(0, no, waits_recv, 0)

    return pl.pallas_call(
        body,
        out_shape=jax.ShapeDtypeStruct((t, d), jnp.float32),
        in_specs=[
            pl.BlockSpec(memory_space=pltpu.SMEM),
            pl.BlockSpec(memory_space=pltpu.MemorySpace.HBM),
        ],
        out_specs=pl.BlockSpec(memory_space=pltpu.VMEM),
        scratch_shapes=[
            pltpu.SMEM((t,), jnp.int32),
            pltpu.SemaphoreType.DMA,
            pltpu.SemaphoreType.DMA,
            pltpu.SemaphoreType.DMA,
        ],
        compiler_params=pltpu.CompilerParams(collective_id=0),
    )(ids, E)


# device time: 32971 ns/iter; 1.4272x vs baseline; 1.4272x over previous
import jax
import jax.numpy as jnp
from jax import lax
from jax.experimental import pallas as pl
from jax.experimental.pallas import tpu as pltpu

V_PER_SHARD = 8192


def kernel(ids, E):
    t = ids.shape[0]
    d = E.shape[1]

    def body(ids_ref, E_ref, out_ref, tok_ref, row_ref, gsem, send_sem, recv_sem):
        x = lax.axis_index("x")
        y = lax.axis_index("y")
        z = lax.axis_index("z")
        partner = x * 8 + (1 - y) * 4 + z
        base = y * V_PER_SHARD

        barrier = pltpu.get_barrier_semaphore()
        pl.semaphore_signal(
            barrier, inc=1, device_id=partner,
            device_id_type=pl.DeviceIdType.LOGICAL,
        )
        pl.semaphore_wait(barrier, 1)

        def scan(tk, nm):
            rid = ids_ref[tk] - base
            owned = rid.astype(jnp.uint32) < V_PER_SHARD

            @pl.when(owned)
            def _():
                tok_ref[nm] = tk
                row_ref[nm] = rid

            return nm + owned.astype(jnp.int32)

        nm = lax.fori_loop(0, t, scan, 0, unroll=8)
        no = t - nm

        def send(j, carry):
            rid = row_ref[j]
            tk = tok_ref[j]
            pltpu.make_async_remote_copy(
                src_ref=E_ref.at[pl.ds(rid, 1)],
                dst_ref=out_ref.at[pl.ds(tk, 1)],
                send_sem=send_sem, recv_sem=recv_sem,
                device_id=partner, device_id_type=pl.DeviceIdType.LOGICAL,
            ).start()
            return carry

        lax.fori_loop(0, nm, send, 0)

        def gather(j, carry):
            rid = row_ref[j]
            tk = tok_ref[j]
            pltpu.make_async_copy(
                E_ref.at[pl.ds(rid, 1)], out_ref.at[pl.ds(tk, 1)], gsem
            ).start()
            return carry

        lax.fori_loop(0, nm, gather, 0)

        src0 = E_ref.at[pl.ds(0, 1)]
        dst0 = out_ref.at[pl.ds(0, 1)]

        def waits_mine(j, carry):
            pltpu.make_async_copy(src0, dst0, gsem).wait()
            pltpu.make_async_remote_copy(
                src_ref=src0, dst_ref=dst0,
                send_sem=send_sem, recv_sem=recv_sem,
                device_id=partner, device_id_type=pl.DeviceIdType.LOGICAL,
            ).wait_send()
            return carry

        def waits_recv(j, carry):
            pltpu.make_async_remote_copy(
                src_ref=src0, dst_ref=dst0,
                send_sem=send_sem, recv_sem=recv_sem,
                device_id=partner, device_id_type=pl.DeviceIdType.LOGICAL,
            ).wait_recv()
            return carry

        lax.fori_loop(0, nm, waits_mine, 0)
        lax.fori_loop(0, no, waits_recv, 0)

    return pl.pallas_call(
        body,
        out_shape=jax.ShapeDtypeStruct((t, d), jnp.float32),
        in_specs=[
            pl.BlockSpec(memory_space=pltpu.SMEM),
            pl.BlockSpec(memory_space=pltpu.MemorySpace.HBM),
        ],
        out_specs=pl.BlockSpec(memory_space=pltpu.VMEM),
        scratch_shapes=[
            pltpu.SMEM((t,), jnp.int32),
            pltpu.SMEM((t,), jnp.int32),
            pltpu.SemaphoreType.DMA,
            pltpu.SemaphoreType.DMA,
            pltpu.SemaphoreType.DMA,
        ],
        compiler_params=pltpu.CompilerParams(collective_id=0),
    )(ids, E)
